# Initial kernel scaffold; baseline (speedup 1.0000x reference)
import jax
import jax.numpy as jnp
from jax.experimental import pallas as pl


def kernel(x, edge_index, W):
    raise NotImplementedError("write your pallas kernel here")


# speedup vs baseline: 109.9498x; 109.9498x over previous
import functools

import jax
import jax.numpy as jnp
from jax import lax
from jax.experimental import pallas as pl
from jax.experimental.pallas import tpu as pltpu
from jax.experimental.pallas import tpu_sc as plsc

_N = 10000
_E = 320000
_D = 128
_NP = 10240
_NC = 2
_NS = 16
_NW = _NC * _NS
_EPW = _E // _NW
_CSL = _NP // _NS
_L = 16

_mesh = plsc.VectorSubcoreMesh(core_axis_name="c", subcore_axis_name="s")
_sc_params = pltpu.CompilerParams(needs_layout_passes=False)


def _zero_vmem(ref, n):
    z = jnp.zeros((_L,), jnp.float32)

    def body(i, carry):
        ref[pl.ds(i * _L, _L)] = z
        return carry

    lax.fori_loop(0, n // _L, body, 0)


def _combine_and_emit(local_v, shared, red_v, out_v, part_hbm, cid, sid):
    pltpu.sync_copy(local_v, shared.at[sid])
    plsc.subcore_barrier()
    for r in range(_NS):
        pltpu.sync_copy(shared.at[r, pl.ds(sid * _CSL, _CSL)], red_v.at[r])

    def comb(j, carry):
        s = red_v[0, pl.ds(j * _L, _L)]
        for r in range(1, _NS):
            s = s + red_v[r, pl.ds(j * _L, _L)]
        out_v[pl.ds(j * _L, _L)] = s
        return carry

    lax.fori_loop(0, _CSL // _L, comb, 0)
    pltpu.sync_copy(out_v, part_hbm.at[cid, pl.ds(sid * _CSL, _CSL)])


@functools.partial(
    pl.kernel,
    out_type=jax.ShapeDtypeStruct((_NC, _NP), jnp.float32),
    mesh=_mesh,
    scratch_types=[
        pltpu.VMEM((_EPW,), jnp.int32),
        pltpu.VMEM((_NP,), jnp.float32),
        pltpu.VMEM_SHARED((_NS, _NP), jnp.float32),
        pltpu.VMEM((_NS, _CSL), jnp.float32),
        pltpu.VMEM((_CSL,), jnp.float32),
    ],
    compiler_params=_sc_params,
)
def _hist(dst_hbm, part_hbm, dst_v, hist_v, shared, red_v, out_v):
    cid = lax.axis_index("c")
    sid = lax.axis_index("s")
    wid = sid * _NC + cid
    _zero_vmem(hist_v, _NP)
    pltpu.sync_copy(dst_hbm.at[pl.ds(wid * _EPW, _EPW)], dst_v)
    one = jnp.ones((_L,), jnp.float32)

    def body(i, carry):
        idx = dst_v[pl.ds(i * _L, _L)]
        plsc.addupdate_scatter(hist_v, [idx], one)
        return carry

    lax.fori_loop(0, _EPW // _L, body, 0)
    _combine_and_emit(hist_v, shared, red_v, out_v, part_hbm, cid, sid)


@functools.partial(
    pl.kernel,
    out_type=jax.ShapeDtypeStruct((_NC, _NP), jnp.float32),
    mesh=_mesh,
    scratch_types=[
        pltpu.VMEM((_NP,), jnp.float32),
        pltpu.VMEM((_EPW,), jnp.int32),
        pltpu.VMEM((_EPW,), jnp.int32),
        pltpu.VMEM((_NP,), jnp.float32),
        pltpu.VMEM_SHARED((_NS, _NP), jnp.float32),
        pltpu.VMEM((_NS, _CSL), jnp.float32),
        pltpu.VMEM((_CSL,), jnp.float32),
    ],
    compiler_params=_sc_params,
)
def _edge_scatter(src_hbm, dst_hbm, g_hbm, part_hbm, g_v, src_v, dst_v,
                  acc_v, shared, red_v, out_v):
    cid = lax.axis_index("c")
    sid = lax.axis_index("s")
    wid = sid * _NC + cid
    _zero_vmem(acc_v, _NP)
    pltpu.sync_copy(g_hbm, g_v)
    pltpu.sync_copy(src_hbm.at[pl.ds(wid * _EPW, _EPW)], src_v)
    pltpu.sync_copy(dst_hbm.at[pl.ds(wid * _EPW, _EPW)], dst_v)

    def body(i, carry):
        sidx = src_v[pl.ds(i * _L, _L)]
        didx = dst_v[pl.ds(i * _L, _L)]
        vals = plsc.load_gather(g_v, [sidx])
        plsc.addupdate_scatter(acc_v, [didx], vals)
        return carry

    lax.fori_loop(0, _EPW // _L, body, 0)
    _combine_and_emit(acc_v, shared, red_v, out_v, part_hbm, cid, sid)


def _dense_body(x_ref, w_ref, degp_ref, y_ref, dinv_ref, g_ref):
    y = jnp.dot(x_ref[...], w_ref[...],
                preferred_element_type=jnp.float32)[:, 0]
    ypad = jnp.concatenate([y, jnp.zeros((_NP - _N,), jnp.float32)])
    deg = degp_ref[0, :] + degp_ref[1, :] + 1.0
    dinv = lax.rsqrt(deg)
    y_ref[...] = ypad
    dinv_ref[...] = dinv
    g_ref[...] = dinv * ypad


_dense = pl.pallas_call(
    _dense_body,
    out_shape=(
        jax.ShapeDtypeStruct((_NP,), jnp.float32),
        jax.ShapeDtypeStruct((_NP,), jnp.float32),
        jax.ShapeDtypeStruct((_NP,), jnp.float32),
    ),
)


def _final_body(accp_ref, dinv_ref, y_ref, out_ref):
    acc = accp_ref[0, :] + accp_ref[1, :]
    dinv = dinv_ref[...]
    out_ref[...] = dinv * (acc + dinv * y_ref[...])


_final = pl.pallas_call(
    _final_body,
    out_shape=jax.ShapeDtypeStruct((_NP,), jnp.float32),
)


def kernel(x, edge_index, W):
    src = edge_index[0]
    dst = edge_index[1]
    deg_part = _hist(dst)
    y, dinv, g = _dense(x, W, deg_part)
    acc_part = _edge_scatter(src, dst, g)
    out = _final(acc_part, dinv, y)
    return out[:_N]


# speedup vs baseline: 149.9473x; 1.3638x over previous
import functools

import jax
import jax.numpy as jnp
from jax import lax
from jax.experimental import pallas as pl
from jax.experimental.pallas import tpu as pltpu
from jax.experimental.pallas import tpu_sc as plsc

_N = 10000
_E = 320000
_D = 128
_NP = 10240
_NC = 2
_NS = 16
_NW = _NC * _NS
_EPW = _E // _NW
_CSL = _NP // _NS
_L = 16
_UNROLL = 5

_mesh = plsc.VectorSubcoreMesh(core_axis_name="c", subcore_axis_name="s")
_sc_params = pltpu.CompilerParams(needs_layout_passes=False)


def _zero_vmem(ref, n):
    z = jnp.zeros((_L,), jnp.float32)

    def body(i, carry):
        ref[pl.ds(i * _L, _L)] = z
        return carry

    lax.fori_loop(0, n // _L, body, 0)


def _combine_and_emit(local_v, shared, red_v, out_v, part_hbm, cid, sid):
    pltpu.sync_copy(local_v, shared.at[sid])
    plsc.subcore_barrier()
    pltpu.sync_copy(shared.at[:, pl.ds(sid * _CSL, _CSL)], red_v)

    def comb(j, carry):
        s = red_v[0, pl.ds(j * _L, _L)]
        for r in range(1, _NS):
            s = s + red_v[r, pl.ds(j * _L, _L)]
        out_v[pl.ds(j * _L, _L)] = s
        return carry

    lax.fori_loop(0, _CSL // _L, comb, 0)
    pltpu.sync_copy(out_v, part_hbm.at[cid, pl.ds(sid * _CSL, _CSL)])


@functools.partial(
    pl.kernel,
    out_type=jax.ShapeDtypeStruct((_NC, _NP), jnp.float32),
    mesh=_mesh,
    scratch_types=[
        pltpu.VMEM((_EPW,), jnp.int32),
        pltpu.VMEM((_NP,), jnp.float32),
        pltpu.VMEM_SHARED((_NS, _NP), jnp.float32),
        pltpu.VMEM((_NS, _CSL), jnp.float32),
        pltpu.VMEM((_CSL,), jnp.float32),
        pltpu.SemaphoreType.DMA,
    ],
    compiler_params=_sc_params,
)
def _hist(edges_hbm, part_hbm, dst_v, hist_v, shared, red_v, out_v, sem):
    cid = lax.axis_index("c")
    sid = lax.axis_index("s")
    wid = sid * _NC + cid
    cp = pltpu.async_copy(edges_hbm.at[pl.ds(_E + wid * _EPW, _EPW)], dst_v,
                          sem)
    _zero_vmem(hist_v, _NP)
    cp.wait()
    one = jnp.ones((_L,), jnp.float32)

    def body(i, carry):
        for u in range(_UNROLL):
            idx = dst_v[pl.ds((i * _UNROLL + u) * _L, _L)]
            plsc.addupdate_scatter(hist_v, [idx], one)
        return carry

    lax.fori_loop(0, _EPW // (_L * _UNROLL), body, 0)
    _combine_and_emit(hist_v, shared, red_v, out_v, part_hbm, cid, sid)


@functools.partial(
    pl.kernel,
    out_type=jax.ShapeDtypeStruct((_NC, _NP), jnp.float32),
    mesh=_mesh,
    scratch_types=[
        pltpu.VMEM((_NP,), jnp.float32),
        pltpu.VMEM((_EPW,), jnp.int32),
        pltpu.VMEM((_EPW,), jnp.int32),
        pltpu.VMEM((_NP,), jnp.float32),
        pltpu.VMEM_SHARED((_NS, _NP), jnp.float32),
        pltpu.VMEM((_NS, _CSL), jnp.float32),
        pltpu.VMEM((_CSL,), jnp.float32),
        pltpu.SemaphoreType.DMA,
    ],
    compiler_params=_sc_params,
)
def _edge_scatter(edges_hbm, g_hbm, part_hbm, g_v, src_v, dst_v, acc_v,
                  shared, red_v, out_v, sem):
    cid = lax.axis_index("c")
    sid = lax.axis_index("s")
    wid = sid * _NC + cid
    cp1 = pltpu.async_copy(g_hbm, g_v, sem)
    cp2 = pltpu.async_copy(edges_hbm.at[pl.ds(wid * _EPW, _EPW)], src_v, sem)
    cp3 = pltpu.async_copy(edges_hbm.at[pl.ds(_E + wid * _EPW, _EPW)], dst_v,
                           sem)
    _zero_vmem(acc_v, _NP)
    cp1.wait()
    cp2.wait()
    cp3.wait()

    def body(i, carry):
        for u in range(_UNROLL):
            off = (i * _UNROLL + u) * _L
            sidx = src_v[pl.ds(off, _L)]
            didx = dst_v[pl.ds(off, _L)]
            vals = plsc.load_gather(g_v, [sidx])
            plsc.addupdate_scatter(acc_v, [didx], vals)
        return carry

    lax.fori_loop(0, _EPW // (_L * _UNROLL), body, 0)
    _combine_and_emit(acc_v, shared, red_v, out_v, part_hbm, cid, sid)


def _dense_body(x_ref, w_ref, degp_ref, y_ref, dinv_ref, g_ref):
    y = jnp.dot(x_ref[...], w_ref[...],
                preferred_element_type=jnp.float32)[:, 0]
    ypad = jnp.concatenate([y, jnp.zeros((_NP - _N,), jnp.float32)])
    deg = degp_ref[0, :] + degp_ref[1, :] + 1.0
    dinv = lax.rsqrt(deg)
    y_ref[...] = ypad
    dinv_ref[...] = dinv
    g_ref[...] = dinv * ypad


_dense = pl.pallas_call(
    _dense_body,
    out_shape=(
        jax.ShapeDtypeStruct((_NP,), jnp.float32),
        jax.ShapeDtypeStruct((_NP,), jnp.float32),
        jax.ShapeDtypeStruct((_NP,), jnp.float32),
    ),
)


def _final_body(accp_ref, dinv_ref, y_ref, out_ref):
    acc = accp_ref[0, :] + accp_ref[1, :]
    dinv = dinv_ref[...]
    out_ref[...] = dinv * (acc + dinv * y_ref[...])


_final = pl.pallas_call(
    _final_body,
    out_shape=jax.ShapeDtypeStruct((_NP,), jnp.float32),
)


def kernel(x, edge_index, W):
    edges = edge_index.reshape(-1)
    deg_part = _hist(edges)
    y, dinv, g = _dense(x, W, deg_part)
    acc_part = _edge_scatter(edges, g)
    out = _final(acc_part, dinv, y)
    return out[:_N]
